# Initial kernel scaffold; baseline (speedup 1.0000x reference)
import jax
import jax.numpy as jnp
from jax.experimental import pallas as pl


def kernel(idx, W):
    raise NotImplementedError("write your pallas kernel here")


# speedup vs baseline: 1.0534x; 1.0534x over previous
import functools

import jax
import jax.numpy as jnp
from jax import lax
from jax.experimental import pallas as pl
from jax.experimental.pallas import tpu as pltpu
from jax.experimental.pallas import tpu_sc as plsc

BATCH = 16384
LANES = 16
NUM_CORES = 2
NUM_SUBCORES = 16
NW = NUM_CORES * NUM_SUBCORES
B_PER_W = BATCH // NW
CHUNK = 128
N_CHUNK = B_PER_W // CHUNK


@functools.partial(
    pl.kernel,
    mesh=plsc.VectorSubcoreMesh(core_axis_name="c", subcore_axis_name="s"),
    out_type=jax.ShapeDtypeStruct((NW, N_CHUNK, CHUNK), jnp.float32),
    scratch_types=[
        pltpu.VMEM((N_CHUNK, CHUNK), jnp.int32),
        pltpu.VMEM((N_CHUNK, CHUNK), jnp.float32),
        pltpu.SemaphoreType.DMA,
    ],
)
def _emb_sigmoid(w_hbm, idx_hbm, out_hbm, idx_v, val_v, sem):
    wid = lax.axis_index("s") * NUM_CORES + lax.axis_index("c")
    pltpu.sync_copy(idx_hbm.at[wid], idx_v)
    copies = [
        pltpu.async_copy(w_hbm.at[idx_v.at[j]], val_v.at[j], sem)
        for j in range(N_CHUNK)
    ]
    for c in copies:
        c.wait()
    for j in range(N_CHUNK):
        for i in range(CHUNK // LANES):
            x = val_v[j, pl.ds(i * LANES, LANES)]
            val_v[j, pl.ds(i * LANES, LANES)] = 1.0 / (1.0 + jnp.exp(-x))
    pltpu.sync_copy(val_v, out_hbm.at[wid])


def kernel(idx, W):
    idx3 = idx.astype(jnp.int32).reshape(NW, N_CHUNK, CHUNK)
    out = _emb_sigmoid(W.reshape(-1), idx3)
    return out.reshape(BATCH, 1)
